# baseline (speedup 1.0000x reference)
import functools

import jax
import jax.numpy as jnp
from jax import lax
from jax.experimental import pallas as pl
from jax.experimental.pallas import tpu as pltpu
from jax.experimental.pallas import tpu_sc as plsc

VOCAB = 100000
EMB = 64
FIX = 50
B = 4096
HIDDEN = 128
OUT = 2

NC = 2
NS = 16
NW = NC * NS

N_IDX = B * FIX
PER_W = N_IDX // NW
SBI = 400
N_SB = PER_W // SBI
CH = 80
N_CH = SBI // CH
OROWS_W = PER_W * EMB // 128
OROWS_SB = SBI * EMB // 128


def _make_sc_gather():
    mesh = plsc.VectorSubcoreMesh(core_axis_name="c", subcore_axis_name="s")

    @functools.partial(
        pl.kernel,
        mesh=mesh,
        out_type=jax.ShapeDtypeStruct((N_IDX, EMB), jnp.float32),
        scratch_types=[
            pltpu.VMEM((PER_W,), jnp.int32),
            pltpu.VMEM((SBI, EMB), jnp.float32),
            pltpu.SemaphoreType.DMA,
        ],
        compiler_params=pltpu.CompilerParams(use_tc_tiling_on_sc=False),
    )
    def gather_k(table_hbm, idx_hbm, out_hbm, idx_v, stage_v, gsem):
        wid = lax.axis_index("s") * NC + lax.axis_index("c")
        pltpu.sync_copy(idx_hbm.at[pl.ds(wid * PER_W, PER_W)], idx_v)

        def body(s, carry):
            descs = [
                pltpu.async_copy(
                    table_hbm.at[idx_v.at[pl.ds(s * SBI + j * CH, CH)]],
                    stage_v.at[pl.ds(j * CH, CH)],
                    gsem,
                )
                for j in range(N_CH)
            ]
            for d in descs:
                d.wait()
            pltpu.sync_copy(
                stage_v,
                out_hbm.at[pl.ds(wid * PER_W + s * SBI, SBI)],
            )
            return carry

        lax.fori_loop(0, N_SB, body, 0)

    return gather_k


_sc_gather = _make_sc_gather()


BM = 256
XROWS = BM * FIX * EMB // 128


KQ = FIX * EMB // 128


def _mlp_body(x_ref, w1_ref, b1_ref, w2_ref, b2_ref, o_ref):
    x3 = x_ref[...].reshape(BM, KQ, 128)
    h = jnp.broadcast_to(b1_ref[...], (BM, HIDDEN))
    for q in range(KQ):
        h = h + jnp.dot(x3[:, q, :], w1_ref[q], preferred_element_type=jnp.float32)
    h = jnp.where(h >= 0, h, 0.01 * h)
    o_ref[...] = jnp.dot(h, w2_ref[...], preferred_element_type=jnp.float32) + b2_ref[...]


def _tc_mlp(rows, W1, b1, W2, b2):
    K = FIX * EMB
    return pl.pallas_call(
        _mlp_body,
        grid=(B // BM,),
        in_specs=[
            pl.BlockSpec((XROWS, 128), lambda i: (i, 0)),
            pl.BlockSpec((KQ, 128, HIDDEN), lambda i: (0, 0, 0)),
            pl.BlockSpec((1, HIDDEN), lambda i: (0, 0)),
            pl.BlockSpec((HIDDEN, OUT), lambda i: (0, 0)),
            pl.BlockSpec((1, OUT), lambda i: (0, 0)),
        ],
        out_specs=pl.BlockSpec((BM, OUT), lambda i: (i, 0)),
        out_shape=jax.ShapeDtypeStruct((B, OUT), jnp.float32),
    )(rows, W1.reshape(KQ, 128, HIDDEN), b1.reshape(1, HIDDEN), W2, b2.reshape(1, OUT))


def kernel(texts, table, W1, b1, W2, b2):
    idx = texts.reshape(N_IDX).astype(jnp.int32)
    rows = _sc_gather(table, idx)
    rows128 = rows.reshape(N_IDX * EMB // 128, 128)
    return _tc_mlp(rows128, W1, b1, W2, b2)


# speedup vs baseline: 1.1396x; 1.1396x over previous
import functools

import jax
import jax.numpy as jnp
from jax import lax
from jax.experimental import pallas as pl
from jax.experimental.pallas import tpu as pltpu
from jax.experimental.pallas import tpu_sc as plsc

VOCAB = 100000
EMB = 64
FIX = 50
B = 4096
HIDDEN = 128
OUT = 2

NC = 2
NS = 16
NW = NC * NS

N_SPLIT = 2
BH = B // N_SPLIT
N_IDX = BH * FIX
PER_W = N_IDX // NW
SBI = 400
N_SB = PER_W // SBI
CH = 80
N_CH = SBI // CH


def _make_sc_gather():
    mesh = plsc.VectorSubcoreMesh(core_axis_name="c", subcore_axis_name="s")

    @functools.partial(
        pl.kernel,
        mesh=mesh,
        out_type=jax.ShapeDtypeStruct((N_IDX, EMB), jnp.float32),
        scratch_types=[
            pltpu.VMEM((PER_W,), jnp.int32),
            pltpu.VMEM((SBI, EMB), jnp.float32),
            pltpu.SemaphoreType.DMA,
        ],
        compiler_params=pltpu.CompilerParams(use_tc_tiling_on_sc=False),
    )
    def gather_k(table_hbm, idx_hbm, out_hbm, idx_v, stage_v, gsem):
        wid = lax.axis_index("s") * NC + lax.axis_index("c")
        pltpu.sync_copy(idx_hbm.at[pl.ds(wid * PER_W, PER_W)], idx_v)

        def body(s, carry):
            descs = [
                pltpu.async_copy(
                    table_hbm.at[idx_v.at[pl.ds(s * SBI + j * CH, CH)]],
                    stage_v.at[pl.ds(j * CH, CH)],
                    gsem,
                )
                for j in range(N_CH)
            ]
            for d in descs:
                d.wait()
            pltpu.sync_copy(
                stage_v,
                out_hbm.at[pl.ds(wid * PER_W + s * SBI, SBI)],
            )
            return carry

        lax.fori_loop(0, N_SB, body, 0)

    return gather_k


_sc_gather = _make_sc_gather()


BM = 256
XROWS = BM * FIX * EMB // 128
K = FIX * EMB


def _mlp_body(x_ref, w1_ref, b1_ref, w2_ref, b2_ref, o_ref):
    x = x_ref[...].reshape(BM, FIX * EMB)
    h = jnp.dot(x, w1_ref[...], preferred_element_type=jnp.float32) + b1_ref[...]
    h = jnp.where(h >= 0, h, 0.01 * h)
    o_ref[...] = jnp.dot(h, w2_ref[...], preferred_element_type=jnp.float32) + b2_ref[...]


def _tc_mlp(rows, W1, b1, W2, b2):
    return pl.pallas_call(
        _mlp_body,
        grid=(BH // BM,),
        in_specs=[
            pl.BlockSpec((XROWS, 128), lambda i: (i, 0)),
            pl.BlockSpec((K, HIDDEN), lambda i: (0, 0)),
            pl.BlockSpec((1, HIDDEN), lambda i: (0, 0)),
            pl.BlockSpec((HIDDEN, OUT), lambda i: (0, 0)),
            pl.BlockSpec((1, OUT), lambda i: (0, 0)),
        ],
        out_specs=pl.BlockSpec((BM, OUT), lambda i: (i, 0)),
        out_shape=jax.ShapeDtypeStruct((BH, OUT), jnp.float32),
    )(rows, W1, b1.reshape(1, HIDDEN), W2, b2.reshape(1, OUT))


def kernel(texts, table, W1, b1, W2, b2):
    idx = texts.reshape(N_SPLIT, N_IDX).astype(jnp.int32)
    b1r = b1.reshape(1, HIDDEN)
    outs = []
    for h in range(N_SPLIT):
        rows = _sc_gather(table, idx[h])
        rows128 = rows.reshape(N_IDX * EMB // 128, 128)
        outs.append(_tc_mlp(rows128, W1, b1, W2, b2))
    return jnp.concatenate(outs, axis=0)
